# baseline (speedup 1.0000x reference)
import functools

import jax
import jax.numpy as jnp
from jax import lax
from jax.experimental import pallas as pl
from jax.experimental.pallas import tpu as pltpu
from jax.experimental.pallas import tpu_sc as plsc

_B = 16384
_D = 32
_NNEG = 20
_NSC = 21
_NC, _NS = 2, 16
_NW = _NC * _NS
_ROWS_W = _B // _NW
_CHUNK = 16
_NCHUNK = _ROWS_W // _CHUNK
_NEG_PER_CHUNK = _CHUNK * _NNEG
_VEC_PER_CHUNK = _CHUNK * _NSC
_OUT_ROWS_PER_CHUNK = _VEC_PER_CHUNK * 16 // 128
_OUT_ROWS = _B * _NSC * 16 // 128


def _sc_partial_body(cW_hbm, xW_hbm, cidx_hbm, pidx_hbm, nidx_hbm, out_hbm,
                     cidx_v, pidx_v, nidx_v, crow_v, prow_v, nrow_v,
                     part_v, sem):
    wid = lax.axis_index("s") * _NC + lax.axis_index("c")
    base = wid * _ROWS_W
    pltpu.sync_copy(cidx_hbm.at[pl.ds(base, _ROWS_W)], cidx_v)
    pltpu.sync_copy(pidx_hbm.at[pl.ds(base, _ROWS_W)], pidx_v)
    pltpu.sync_copy(nidx_hbm.at[pl.ds(base * _NNEG, _ROWS_W * _NNEG)], nidx_v)

    def chunk_body(ci, carry):
        rbase = ci * _CHUNK
        nbase = ci * _NEG_PER_CHUNK
        dmas = [
            pltpu.async_copy(
                cW_hbm.at[cidx_v.at[pl.ds(rbase, _CHUNK)]], crow_v, sem),
            pltpu.async_copy(
                xW_hbm.at[pidx_v.at[pl.ds(rbase, _CHUNK)]], prow_v, sem),
        ]
        for g, (off, ln) in enumerate(((0, 128), (128, 128), (256, 64))):
            dmas.append(pltpu.async_copy(
                xW_hbm.at[nidx_v.at[pl.ds(nbase + off, ln)]],
                nrow_v.at[pl.ds(off, ln)], sem))
        for d in dmas:
            d.wait()

        def row_body(r, rcarry):
            c_lo = crow_v[r, pl.ds(0, 16)]
            c_hi = crow_v[r, pl.ds(16, 16)]
            ncl, nch = -c_lo, -c_hi
            x_lo = prow_v[r, pl.ds(0, 16)]
            x_hi = prow_v[r, pl.ds(16, 16)]
            part_v[r // 8, pl.ds((r % 8) * 16, 16)] = c_lo * x_lo + c_hi * x_hi
            for j in range(_NNEG):
                k = r * _NNEG + j
                n_lo = nrow_v[k, pl.ds(0, 16)]
                n_hi = nrow_v[k, pl.ds(16, 16)]
                part_v[2 + k // 8, pl.ds((k % 8) * 16, 16)] = (
                    ncl * n_lo + nch * n_hi)
            return rcarry

        lax.fori_loop(0, _CHUNK, row_body, 0)
        pltpu.sync_copy(
            part_v,
            out_hbm.at[pl.ds((wid * _NCHUNK + ci) * _OUT_ROWS_PER_CHUNK,
                             _OUT_ROWS_PER_CHUNK)])
        return carry

    lax.fori_loop(0, _NCHUNK, chunk_body, 0)


_sc_partial = functools.partial(
    pl.kernel,
    mesh=plsc.VectorSubcoreMesh(core_axis_name="c", subcore_axis_name="s"),
    out_type=jax.ShapeDtypeStruct((_OUT_ROWS, 128), jnp.float32),
    scratch_types=[
        pltpu.VMEM((_ROWS_W,), jnp.int32),
        pltpu.VMEM((_ROWS_W,), jnp.int32),
        pltpu.VMEM((_ROWS_W * _NNEG,), jnp.int32),
        pltpu.VMEM((_CHUNK, _D), jnp.float32),
        pltpu.VMEM((_CHUNK, _D), jnp.float32),
        pltpu.VMEM((_NEG_PER_CHUNK, _D), jnp.float32),
        pltpu.VMEM((_OUT_ROWS_PER_CHUNK, 128), jnp.float32),
        pltpu.SemaphoreType.DMA,
    ],
    compiler_params=pltpu.CompilerParams(use_tc_tiling_on_sc=False),
)(_sc_partial_body)

_CV = 512
_CGRID = (1000000 + _CV - 1) // _CV


def _conv_body(a_ref, b_ref, oa_ref, ob_ref):
    c = lax.broadcasted_iota(jnp.int32, (_CV, _CV // 4), 0)
    r = lax.broadcasted_iota(jnp.int32, (_CV, _CV // 4), 1)
    for (x, o_ref) in ((a_ref[...], oa_ref), (b_ref[...], ob_ref)):
        x = jnp.where(jnp.isfinite(x), x, 0.0)
        for g in range(4):
            sel = (c == 4 * r + g).astype(jnp.float32)
            og = lax.dot_general(sel, x, (((0,), (1,)), ((), ())),
                                 preferred_element_type=jnp.float32)
            o_ref[:, pl.ds(32 * g, 32)] = og


_conv_call = pl.pallas_call(
    _conv_body,
    grid=(_CGRID,),
    in_specs=[pl.BlockSpec((32, _CV), lambda i: (0, i)),
              pl.BlockSpec((32, _CV), lambda i: (0, i))],
    out_specs=[pl.BlockSpec((_CV // 4, 128), lambda i: (i, 0)),
               pl.BlockSpec((_CV // 4, 128), lambda i: (i, 0))],
    out_shape=[jax.ShapeDtypeStruct((250000, 128), jnp.float32),
               jax.ShapeDtypeStruct((250000, 128), jnp.float32)],
)

_BLK = 7168
_NBLK = _OUT_ROWS // _BLK


def _loss_body(p_ref, o_ref):
    i = pl.program_id(0)
    x = p_ref[...]
    lane = lax.broadcasted_iota(jnp.int32, (128, 8), 0)
    grp = lax.broadcasted_iota(jnp.int32, (128, 8), 1)
    m = jnp.where(lane // 16 == grp, 1.0, 0.0).astype(jnp.float32)
    s = jnp.dot(x, m, preferred_element_type=jnp.float32)
    ls = jnp.minimum(s, 0.0) - jnp.log1p(jnp.exp(-jnp.abs(s)))

    @pl.when(i == 0)
    def _init():
        o_ref[...] = jnp.zeros((1, 1), jnp.float32)

    o_ref[...] += jnp.sum(ls).reshape(1, 1)

    @pl.when(i == _NBLK - 1)
    def _fini():
        o_ref[...] = -o_ref[...] / _B


_loss_call = pl.pallas_call(
    _loss_body,
    grid=(_NBLK,),
    in_specs=[pl.BlockSpec((_BLK, 128), lambda i: (i, 0))],
    out_specs=pl.BlockSpec((1, 1), lambda i: (0, 0)),
    out_shape=jax.ShapeDtypeStruct((1, 1), jnp.float32),
)


def kernel(center, context, negatives, center_W, context_W):
    cidx = center.reshape(_B).astype(jnp.int32)
    pidx = context.reshape(_B).astype(jnp.int32)
    nidx = negatives.reshape(_B * _NNEG).astype(jnp.int32)
    cw4, xw4 = _conv_call(center_W.T, context_W.T)
    part = _sc_partial(cw4.reshape(1000000, 32), xw4.reshape(1000000, 32),
                       cidx, pidx, nidx)
    return _loss_call(part).reshape(())


# speedup vs baseline: 2.2147x; 2.2147x over previous
import functools

import jax
import jax.numpy as jnp
from jax import lax
from jax.experimental import pallas as pl
from jax.experimental.pallas import tpu as pltpu
from jax.experimental.pallas import tpu_sc as plsc

_B = 16384
_D = 32
_NNEG = 20
_NSC = 21
_NC, _NS = 2, 16
_NW = _NC * _NS
_ROWS_W = _B // _NW
_CHUNK = 16
_NCHUNK = _ROWS_W // _CHUNK
_NEG_PER_CHUNK = _CHUNK * _NNEG
_VEC_PER_CHUNK = _CHUNK * _NSC
_OUT_ROWS_PER_CHUNK = _VEC_PER_CHUNK * 16 // 128
_OUT_ROWS = _B * _NSC * 16 // 128

_CV = 2048
_CQ = _CV // 4
_CGRID = (1000000 + _CV - 1) // _CV
_VPAD = _CGRID * _CV


def _conv_body(a_ref, b_ref, oa_ref, ob_ref):
    for (in_ref, o_ref) in ((a_ref, oa_ref), (b_ref, ob_ref)):
        xt = jnp.transpose(in_ref[...])
        for g in range(4):
            o_ref[:, pl.ds(32 * g, 32)] = xt[_CQ * g:_CQ * (g + 1), :]


_conv_call = pl.pallas_call(
    _conv_body,
    grid=(_CGRID,),
    in_specs=[pl.BlockSpec((32, _CV), lambda i: (0, i)),
              pl.BlockSpec((32, _CV), lambda i: (0, i))],
    out_specs=[pl.BlockSpec((_CQ, 128), lambda i: (i, 0)),
               pl.BlockSpec((_CQ, 128), lambda i: (i, 0))],
    out_shape=[jax.ShapeDtypeStruct((_VPAD // 4, 128), jnp.float32),
               jax.ShapeDtypeStruct((_VPAD // 4, 128), jnp.float32)],
)


def _permute_idx(ref, nvec):
    def body(k, carry):
        v = ref[pl.ds(k * 16, 16)]
        c = jnp.bitwise_and(v, _CV - 1)
        r = jnp.bitwise_and(c, _CQ - 1)
        g = jnp.right_shift(c, 9)
        ref[pl.ds(k * 16, 16)] = (v - c) + jnp.left_shift(r, 2) + g
        return carry

    lax.fori_loop(0, nvec, body, 0)


def _sc_partial_body(cW_hbm, xW_hbm, cidx_hbm, pidx_hbm, nidx_hbm, out_hbm,
                     cidx_v, pidx_v, nidx_v, crow_v, prow_v, nrow_v,
                     part_v, sem):
    wid = lax.axis_index("s") * _NC + lax.axis_index("c")
    base = wid * _ROWS_W
    pltpu.sync_copy(cidx_hbm.at[pl.ds(base, _ROWS_W)], cidx_v)
    pltpu.sync_copy(pidx_hbm.at[pl.ds(base, _ROWS_W)], pidx_v)
    pltpu.sync_copy(nidx_hbm.at[pl.ds(base * _NNEG, _ROWS_W * _NNEG)], nidx_v)
    _permute_idx(cidx_v, _ROWS_W // 16)
    _permute_idx(pidx_v, _ROWS_W // 16)
    _permute_idx(nidx_v, _ROWS_W * _NNEG // 16)

    def chunk_body(ci, carry):
        rbase = ci * _CHUNK
        nbase = ci * _NEG_PER_CHUNK
        dmas = [
            pltpu.async_copy(
                cW_hbm.at[cidx_v.at[pl.ds(rbase, _CHUNK)]], crow_v, sem),
            pltpu.async_copy(
                xW_hbm.at[pidx_v.at[pl.ds(rbase, _CHUNK)]], prow_v, sem),
        ]
        for off, ln in ((0, 128), (128, 128), (256, 64)):
            dmas.append(pltpu.async_copy(
                xW_hbm.at[nidx_v.at[pl.ds(nbase + off, ln)]],
                nrow_v.at[pl.ds(off, ln)], sem))
        for d in dmas:
            d.wait()

        def row_body(r, rcarry):
            c_lo = crow_v[r, pl.ds(0, 16)]
            c_hi = crow_v[r, pl.ds(16, 16)]
            ncl, nch = -c_lo, -c_hi
            x_lo = prow_v[r, pl.ds(0, 16)]
            x_hi = prow_v[r, pl.ds(16, 16)]
            part_v[r // 8, pl.ds((r % 8) * 16, 16)] = c_lo * x_lo + c_hi * x_hi
            for j in range(_NNEG):
                k = r * _NNEG + j
                n_lo = nrow_v[k, pl.ds(0, 16)]
                n_hi = nrow_v[k, pl.ds(16, 16)]
                part_v[2 + k // 8, pl.ds((k % 8) * 16, 16)] = (
                    ncl * n_lo + nch * n_hi)
            return rcarry

        lax.fori_loop(0, _CHUNK, row_body, 0)
        pltpu.sync_copy(
            part_v,
            out_hbm.at[pl.ds((wid * _NCHUNK + ci) * _OUT_ROWS_PER_CHUNK,
                             _OUT_ROWS_PER_CHUNK)])
        return carry

    lax.fori_loop(0, _NCHUNK, chunk_body, 0)


_sc_partial = functools.partial(
    pl.kernel,
    mesh=plsc.VectorSubcoreMesh(core_axis_name="c", subcore_axis_name="s"),
    out_type=jax.ShapeDtypeStruct((_OUT_ROWS, 128), jnp.float32),
    scratch_types=[
        pltpu.VMEM((_ROWS_W,), jnp.int32),
        pltpu.VMEM((_ROWS_W,), jnp.int32),
        pltpu.VMEM((_ROWS_W * _NNEG,), jnp.int32),
        pltpu.VMEM((_CHUNK, _D), jnp.float32),
        pltpu.VMEM((_CHUNK, _D), jnp.float32),
        pltpu.VMEM((_NEG_PER_CHUNK, _D), jnp.float32),
        pltpu.VMEM((_OUT_ROWS_PER_CHUNK, 128), jnp.float32),
        pltpu.SemaphoreType.DMA,
    ],
    compiler_params=pltpu.CompilerParams(use_tc_tiling_on_sc=False),
)(_sc_partial_body)

_BLK = 7168
_NBLK = _OUT_ROWS // _BLK


def _loss_body(p_ref, o_ref):
    i = pl.program_id(0)
    x = p_ref[...]
    lane = lax.broadcasted_iota(jnp.int32, (128, 8), 0)
    grp = lax.broadcasted_iota(jnp.int32, (128, 8), 1)
    m = jnp.where(lane // 16 == grp, 1.0, 0.0).astype(jnp.float32)
    s = jnp.dot(x, m, preferred_element_type=jnp.float32)
    ls = jnp.minimum(s, 0.0) - jnp.log1p(jnp.exp(-jnp.abs(s)))

    @pl.when(i == 0)
    def _init():
        o_ref[...] = jnp.zeros((1, 1), jnp.float32)

    o_ref[...] += jnp.sum(ls).reshape(1, 1)

    @pl.when(i == _NBLK - 1)
    def _fini():
        o_ref[...] = -o_ref[...] / _B


_loss_call = pl.pallas_call(
    _loss_body,
    grid=(_NBLK,),
    in_specs=[pl.BlockSpec((_BLK, 128), lambda i: (i, 0))],
    out_specs=pl.BlockSpec((1, 1), lambda i: (0, 0)),
    out_shape=jax.ShapeDtypeStruct((1, 1), jnp.float32),
)


def kernel(center, context, negatives, center_W, context_W):
    cidx = center.reshape(_B).astype(jnp.int32)
    pidx = context.reshape(_B).astype(jnp.int32)
    nidx = negatives.reshape(_B * _NNEG).astype(jnp.int32)
    cw4, xw4 = _conv_call(center_W.T, context_W.T)
    part = _sc_partial(cw4.reshape(_VPAD, 32), xw4.reshape(_VPAD, 32),
                       cidx, pidx, nidx)
    return _loss_call(part).reshape(())


# speedup vs baseline: 2.2196x; 1.0022x over previous
import functools

import jax
import jax.numpy as jnp
from jax import lax
from jax.experimental import pallas as pl
from jax.experimental.pallas import tpu as pltpu
from jax.experimental.pallas import tpu_sc as plsc

_B = 16384
_D = 32
_NNEG = 20
_NSC = 21
_NC, _NS = 2, 16
_NW = _NC * _NS
_ROWS_W = _B // _NW
_CHUNK = 16
_NCHUNK = _ROWS_W // _CHUNK
_NEG_PER_CHUNK = _CHUNK * _NNEG
_VEC_PER_CHUNK = _CHUNK * _NSC
_OUT_ROWS_PER_CHUNK = _VEC_PER_CHUNK * 16 // 128
_OUT_ROWS = _B * _NSC * 16 // 128

_CV = 2048
_CQ = _CV // 4
_CGRID = (1000000 + _CV - 1) // _CV
_VPAD = _CGRID * _CV


def _conv_body(a_ref, b_ref, oa_ref, ob_ref):
    for (in_ref, o_ref) in ((a_ref, oa_ref), (b_ref, ob_ref)):
        for g in range(4):
            xg = in_ref[:, pl.ds(_CQ * g, _CQ)]
            o_ref[:, pl.ds(32 * g, 32)] = jnp.transpose(xg)


_conv_call = pl.pallas_call(
    _conv_body,
    grid=(_CGRID,),
    in_specs=[pl.BlockSpec((32, _CV), lambda i: (0, i)),
              pl.BlockSpec((32, _CV), lambda i: (0, i))],
    out_specs=[pl.BlockSpec((_CQ, 128), lambda i: (i, 0)),
               pl.BlockSpec((_CQ, 128), lambda i: (i, 0))],
    out_shape=[jax.ShapeDtypeStruct((_VPAD // 4, 128), jnp.float32),
               jax.ShapeDtypeStruct((_VPAD // 4, 128), jnp.float32)],
)


def _permute_idx(ref, nvec):
    def body(k, carry):
        v = ref[pl.ds(k * 16, 16)]
        c = jnp.bitwise_and(v, _CV - 1)
        r = jnp.bitwise_and(c, _CQ - 1)
        g = jnp.right_shift(c, 9)
        ref[pl.ds(k * 16, 16)] = (v - c) + jnp.left_shift(r, 2) + g
        return carry

    lax.fori_loop(0, nvec, body, 0)


def _sc_partial_body(cW_hbm, xW_hbm, cidx_hbm, pidx_hbm, nidx_hbm, out_hbm,
                     cidx_v, pidx_v, nidx_v, crow_v, prow_v, nrow_v,
                     part_v, sem):
    wid = lax.axis_index("s") * _NC + lax.axis_index("c")
    base = wid * _ROWS_W
    pltpu.sync_copy(cidx_hbm.at[pl.ds(base, _ROWS_W)], cidx_v)
    pltpu.sync_copy(pidx_hbm.at[pl.ds(base, _ROWS_W)], pidx_v)
    pltpu.sync_copy(nidx_hbm.at[pl.ds(base * _NNEG, _ROWS_W * _NNEG)], nidx_v)
    _permute_idx(cidx_v, _ROWS_W // 16)
    _permute_idx(pidx_v, _ROWS_W // 16)
    _permute_idx(nidx_v, _ROWS_W * _NNEG // 16)

    def chunk_body(ci, carry):
        rbase = ci * _CHUNK
        nbase = ci * _NEG_PER_CHUNK
        dmas = [
            pltpu.async_copy(
                cW_hbm.at[cidx_v.at[pl.ds(rbase, _CHUNK)]], crow_v, sem),
            pltpu.async_copy(
                xW_hbm.at[pidx_v.at[pl.ds(rbase, _CHUNK)]], prow_v, sem),
        ]
        for off, ln in ((0, 128), (128, 128), (256, 64)):
            dmas.append(pltpu.async_copy(
                xW_hbm.at[nidx_v.at[pl.ds(nbase + off, ln)]],
                nrow_v.at[pl.ds(off, ln)], sem))
        for d in dmas:
            d.wait()

        def row_body(r, rcarry):
            c_lo = crow_v[r, pl.ds(0, 16)]
            c_hi = crow_v[r, pl.ds(16, 16)]
            ncl, nch = -c_lo, -c_hi
            x_lo = prow_v[r, pl.ds(0, 16)]
            x_hi = prow_v[r, pl.ds(16, 16)]
            part_v[r // 8, pl.ds((r % 8) * 16, 16)] = c_lo * x_lo + c_hi * x_hi
            for j in range(_NNEG):
                k = r * _NNEG + j
                n_lo = nrow_v[k, pl.ds(0, 16)]
                n_hi = nrow_v[k, pl.ds(16, 16)]
                part_v[2 + k // 8, pl.ds((k % 8) * 16, 16)] = (
                    ncl * n_lo + nch * n_hi)
            return rcarry

        lax.fori_loop(0, _CHUNK, row_body, 0)
        pltpu.sync_copy(
            part_v,
            out_hbm.at[pl.ds((wid * _NCHUNK + ci) * _OUT_ROWS_PER_CHUNK,
                             _OUT_ROWS_PER_CHUNK)])
        return carry

    lax.fori_loop(0, _NCHUNK, chunk_body, 0)


_sc_partial = functools.partial(
    pl.kernel,
    mesh=plsc.VectorSubcoreMesh(core_axis_name="c", subcore_axis_name="s"),
    out_type=jax.ShapeDtypeStruct((_OUT_ROWS, 128), jnp.float32),
    scratch_types=[
        pltpu.VMEM((_ROWS_W,), jnp.int32),
        pltpu.VMEM((_ROWS_W,), jnp.int32),
        pltpu.VMEM((_ROWS_W * _NNEG,), jnp.int32),
        pltpu.VMEM((_CHUNK, _D), jnp.float32),
        pltpu.VMEM((_CHUNK, _D), jnp.float32),
        pltpu.VMEM((_NEG_PER_CHUNK, _D), jnp.float32),
        pltpu.VMEM((_OUT_ROWS_PER_CHUNK, 128), jnp.float32),
        pltpu.SemaphoreType.DMA,
    ],
    compiler_params=pltpu.CompilerParams(use_tc_tiling_on_sc=False),
)(_sc_partial_body)

_BLK = 7168
_NBLK = _OUT_ROWS // _BLK


def _loss_body(p_ref, o_ref):
    i = pl.program_id(0)
    x = p_ref[...]
    lane = lax.broadcasted_iota(jnp.int32, (128, 8), 0)
    grp = lax.broadcasted_iota(jnp.int32, (128, 8), 1)
    m = jnp.where(lane // 16 == grp, 1.0, 0.0).astype(jnp.float32)
    s = jnp.dot(x, m, preferred_element_type=jnp.float32)
    ls = jnp.minimum(s, 0.0) - jnp.log1p(jnp.exp(-jnp.abs(s)))

    @pl.when(i == 0)
    def _init():
        o_ref[...] = jnp.zeros((1, 1), jnp.float32)

    o_ref[...] += jnp.sum(ls).reshape(1, 1)

    @pl.when(i == _NBLK - 1)
    def _fini():
        o_ref[...] = -o_ref[...] / _B


_loss_call = pl.pallas_call(
    _loss_body,
    grid=(_NBLK,),
    in_specs=[pl.BlockSpec((_BLK, 128), lambda i: (i, 0))],
    out_specs=pl.BlockSpec((1, 1), lambda i: (0, 0)),
    out_shape=jax.ShapeDtypeStruct((1, 1), jnp.float32),
)


def kernel(center, context, negatives, center_W, context_W):
    cidx = center.reshape(_B).astype(jnp.int32)
    pidx = context.reshape(_B).astype(jnp.int32)
    nidx = negatives.reshape(_B * _NNEG).astype(jnp.int32)
    cw4, xw4 = _conv_call(center_W.T, context_W.T)
    part = _sc_partial(cw4.reshape(_VPAD, 32), xw4.reshape(_VPAD, 32),
                       cidx, pidx, nidx)
    return _loss_call(part).reshape(())


# speedup vs baseline: 2.5268x; 1.1384x over previous
import functools

import jax
import jax.numpy as jnp
from jax import lax
from jax.experimental import pallas as pl
from jax.experimental.pallas import tpu as pltpu
from jax.experimental.pallas import tpu_sc as plsc

_B = 16384
_D = 32
_NNEG = 20
_NSC = 21
_NC, _NS = 2, 16
_NW = _NC * _NS
_ROWS_W = _B // _NW
_CHUNK = 16
_NCHUNK = _ROWS_W // _CHUNK
_NEG_PER_CHUNK = _CHUNK * _NNEG
_VEC_PER_CHUNK = _CHUNK * _NSC
_OUT_ROWS_PER_CHUNK = _VEC_PER_CHUNK * 16 // 128
_OUT_ROWS = _B * _NSC * 16 // 128

_CV = 4096
_CQ = _CV // 4
_CGRID = (1000000 + _CV - 1) // _CV
_VPAD = _CGRID * _CV


def _conv_body(a_ref, b_ref, oa_ref, ob_ref):
    for (in_ref, o_ref) in ((a_ref, oa_ref), (b_ref, ob_ref)):
        o_ref[...] = jnp.concatenate(
            [jnp.transpose(in_ref[:, pl.ds(_CQ * g, _CQ)]) for g in range(4)],
            axis=1)


_conv_call = pl.pallas_call(
    _conv_body,
    grid=(_CGRID,),
    in_specs=[pl.BlockSpec((32, _CV), lambda i: (0, i)),
              pl.BlockSpec((32, _CV), lambda i: (0, i))],
    out_specs=[pl.BlockSpec((_CQ, 128), lambda i: (i, 0)),
               pl.BlockSpec((_CQ, 128), lambda i: (i, 0))],
    out_shape=[jax.ShapeDtypeStruct((_VPAD // 4, 128), jnp.float32),
               jax.ShapeDtypeStruct((_VPAD // 4, 128), jnp.float32)],
)


def _permute_idx(ref, nvec):
    def body(k, carry):
        v = ref[pl.ds(k * 16, 16)]
        c = jnp.bitwise_and(v, _CV - 1)
        r = jnp.bitwise_and(c, _CQ - 1)
        g = jnp.right_shift(c, _CQ.bit_length() - 1)
        ref[pl.ds(k * 16, 16)] = (v - c) + jnp.left_shift(r, 2) + g
        return carry

    lax.fori_loop(0, nvec, body, 0)


def _sc_partial_body(cW_hbm, xW_hbm, cidx_hbm, pidx_hbm, nidx_hbm, out_hbm,
                     cidx_v, pidx_v, nidx_v, crow_v, prow_v, nrow_v,
                     part_v, sem):
    wid = lax.axis_index("s") * _NC + lax.axis_index("c")
    base = wid * _ROWS_W
    pltpu.sync_copy(cidx_hbm.at[pl.ds(base, _ROWS_W)], cidx_v)
    pltpu.sync_copy(pidx_hbm.at[pl.ds(base, _ROWS_W)], pidx_v)
    pltpu.sync_copy(nidx_hbm.at[pl.ds(base * _NNEG, _ROWS_W * _NNEG)], nidx_v)
    _permute_idx(cidx_v, _ROWS_W // 16)
    _permute_idx(pidx_v, _ROWS_W // 16)
    _permute_idx(nidx_v, _ROWS_W * _NNEG // 16)

    def chunk_body(ci, carry):
        rbase = ci * _CHUNK
        nbase = ci * _NEG_PER_CHUNK
        dmas = [
            pltpu.async_copy(
                cW_hbm.at[cidx_v.at[pl.ds(rbase, _CHUNK)]], crow_v, sem),
            pltpu.async_copy(
                xW_hbm.at[pidx_v.at[pl.ds(rbase, _CHUNK)]], prow_v, sem),
        ]
        for off, ln in ((0, 128), (128, 128), (256, 64)):
            dmas.append(pltpu.async_copy(
                xW_hbm.at[nidx_v.at[pl.ds(nbase + off, ln)]],
                nrow_v.at[pl.ds(off, ln)], sem))
        for d in dmas:
            d.wait()

        def row_body(r, rcarry):
            c_lo = crow_v[r, pl.ds(0, 16)]
            c_hi = crow_v[r, pl.ds(16, 16)]
            ncl, nch = -c_lo, -c_hi
            x_lo = prow_v[r, pl.ds(0, 16)]
            x_hi = prow_v[r, pl.ds(16, 16)]
            part_v[r // 8, pl.ds((r % 8) * 16, 16)] = c_lo * x_lo + c_hi * x_hi
            for j in range(_NNEG):
                k = r * _NNEG + j
                n_lo = nrow_v[k, pl.ds(0, 16)]
                n_hi = nrow_v[k, pl.ds(16, 16)]
                part_v[2 + k // 8, pl.ds((k % 8) * 16, 16)] = (
                    ncl * n_lo + nch * n_hi)
            return rcarry

        lax.fori_loop(0, _CHUNK, row_body, 0)
        pltpu.sync_copy(
            part_v,
            out_hbm.at[pl.ds((wid * _NCHUNK + ci) * _OUT_ROWS_PER_CHUNK,
                             _OUT_ROWS_PER_CHUNK)])
        return carry

    lax.fori_loop(0, _NCHUNK, chunk_body, 0)


_sc_partial = functools.partial(
    pl.kernel,
    mesh=plsc.VectorSubcoreMesh(core_axis_name="c", subcore_axis_name="s"),
    out_type=jax.ShapeDtypeStruct((_OUT_ROWS, 128), jnp.float32),
    scratch_types=[
        pltpu.VMEM((_ROWS_W,), jnp.int32),
        pltpu.VMEM((_ROWS_W,), jnp.int32),
        pltpu.VMEM((_ROWS_W * _NNEG,), jnp.int32),
        pltpu.VMEM((_CHUNK, _D), jnp.float32),
        pltpu.VMEM((_CHUNK, _D), jnp.float32),
        pltpu.VMEM((_NEG_PER_CHUNK, _D), jnp.float32),
        pltpu.VMEM((_OUT_ROWS_PER_CHUNK, 128), jnp.float32),
        pltpu.SemaphoreType.DMA,
    ],
    compiler_params=pltpu.CompilerParams(use_tc_tiling_on_sc=False),
)(_sc_partial_body)

_BLK = 7168
_NBLK = _OUT_ROWS // _BLK


def _loss_body(p_ref, o_ref):
    i = pl.program_id(0)
    x = p_ref[...]
    lane = lax.broadcasted_iota(jnp.int32, (128, 8), 0)
    grp = lax.broadcasted_iota(jnp.int32, (128, 8), 1)
    m = jnp.where(lane // 16 == grp, 1.0, 0.0).astype(jnp.float32)
    s = jnp.dot(x, m, preferred_element_type=jnp.float32)
    ls = jnp.minimum(s, 0.0) - jnp.log1p(jnp.exp(-jnp.abs(s)))

    @pl.when(i == 0)
    def _init():
        o_ref[...] = jnp.zeros((1, 1), jnp.float32)

    o_ref[...] += jnp.sum(ls).reshape(1, 1)

    @pl.when(i == _NBLK - 1)
    def _fini():
        o_ref[...] = -o_ref[...] / _B


_loss_call = pl.pallas_call(
    _loss_body,
    grid=(_NBLK,),
    in_specs=[pl.BlockSpec((_BLK, 128), lambda i: (i, 0))],
    out_specs=pl.BlockSpec((1, 1), lambda i: (0, 0)),
    out_shape=jax.ShapeDtypeStruct((1, 1), jnp.float32),
)


def kernel(center, context, negatives, center_W, context_W):
    cidx = center.reshape(_B).astype(jnp.int32)
    pidx = context.reshape(_B).astype(jnp.int32)
    nidx = negatives.reshape(_B * _NNEG).astype(jnp.int32)
    cw4, xw4 = _conv_call(center_W.T, context_W.T)
    part = _sc_partial(cw4.reshape(_VPAD, 32), xw4.reshape(_VPAD, 32),
                       cidx, pidx, nidx)
    return _loss_call(part).reshape(())


# speedup vs baseline: 2.5744x; 1.0188x over previous
import functools

import jax
import jax.numpy as jnp
from jax import lax
from jax.experimental import pallas as pl
from jax.experimental.pallas import tpu as pltpu
from jax.experimental.pallas import tpu_sc as plsc

_B = 16384
_D = 32
_NNEG = 20
_NSC = 21
_NC, _NS = 2, 16
_NW = _NC * _NS
_ROWS_W = _B // _NW
_CHUNK = 16
_NCHUNK = _ROWS_W // _CHUNK
_NEG_PER_CHUNK = _CHUNK * _NNEG
_VEC_PER_CHUNK = _CHUNK * _NSC
_OUT_ROWS_PER_CHUNK = _VEC_PER_CHUNK * 16 // 128
_OUT_ROWS = _B * _NSC * 16 // 128

_CV = 8192
_CQ = _CV // 4
_CGRID = (1000000 + _CV - 1) // _CV
_VPAD = _CGRID * _CV


def _conv_body(a_ref, b_ref, oa_ref, ob_ref):
    for (in_ref, o_ref) in ((a_ref, oa_ref), (b_ref, ob_ref)):
        o_ref[...] = jnp.concatenate(
            [jnp.transpose(in_ref[:, pl.ds(_CQ * g, _CQ)]) for g in range(4)],
            axis=1)


_conv_call = pl.pallas_call(
    _conv_body,
    grid=(_CGRID,),
    in_specs=[pl.BlockSpec((32, _CV), lambda i: (0, i)),
              pl.BlockSpec((32, _CV), lambda i: (0, i))],
    out_specs=[pl.BlockSpec((_CQ, 128), lambda i: (i, 0)),
               pl.BlockSpec((_CQ, 128), lambda i: (i, 0))],
    out_shape=[jax.ShapeDtypeStruct((_VPAD // 4, 128), jnp.float32),
               jax.ShapeDtypeStruct((_VPAD // 4, 128), jnp.float32)],
)


def _permute_idx(ref, nvec):
    def body(k, carry):
        v = ref[pl.ds(k * 16, 16)]
        c = jnp.bitwise_and(v, _CV - 1)
        r = jnp.bitwise_and(c, _CQ - 1)
        g = jnp.right_shift(c, _CQ.bit_length() - 1)
        ref[pl.ds(k * 16, 16)] = (v - c) + jnp.left_shift(r, 2) + g
        return carry

    lax.fori_loop(0, nvec, body, 0)


def _sc_partial_body(cW_hbm, xW_hbm, cidx_hbm, pidx_hbm, nidx_hbm, out_hbm,
                     cidx_v, pidx_v, nidx_v, crow_v, prow_v, nrow_v,
                     part_v, sem):
    wid = lax.axis_index("s") * _NC + lax.axis_index("c")
    base = wid * _ROWS_W
    pltpu.sync_copy(cidx_hbm.at[pl.ds(base, _ROWS_W)], cidx_v)
    pltpu.sync_copy(pidx_hbm.at[pl.ds(base, _ROWS_W)], pidx_v)
    pltpu.sync_copy(nidx_hbm.at[pl.ds(base * _NNEG, _ROWS_W * _NNEG)], nidx_v)
    _permute_idx(cidx_v, _ROWS_W // 16)
    _permute_idx(pidx_v, _ROWS_W // 16)
    _permute_idx(nidx_v, _ROWS_W * _NNEG // 16)

    def chunk_body(ci, carry):
        rbase = ci * _CHUNK
        nbase = ci * _NEG_PER_CHUNK
        dmas = [
            pltpu.async_copy(
                cW_hbm.at[cidx_v.at[pl.ds(rbase, _CHUNK)]], crow_v, sem),
            pltpu.async_copy(
                xW_hbm.at[pidx_v.at[pl.ds(rbase, _CHUNK)]], prow_v, sem),
        ]
        for off, ln in ((0, 128), (128, 128), (256, 64)):
            dmas.append(pltpu.async_copy(
                xW_hbm.at[nidx_v.at[pl.ds(nbase + off, ln)]],
                nrow_v.at[pl.ds(off, ln)], sem))
        for d in dmas:
            d.wait()

        def row_body(r, rcarry):
            c_lo = crow_v[r, pl.ds(0, 16)]
            c_hi = crow_v[r, pl.ds(16, 16)]
            ncl, nch = -c_lo, -c_hi
            x_lo = prow_v[r, pl.ds(0, 16)]
            x_hi = prow_v[r, pl.ds(16, 16)]
            part_v[r // 8, pl.ds((r % 8) * 16, 16)] = c_lo * x_lo + c_hi * x_hi
            for j in range(_NNEG):
                k = r * _NNEG + j
                n_lo = nrow_v[k, pl.ds(0, 16)]
                n_hi = nrow_v[k, pl.ds(16, 16)]
                part_v[2 + k // 8, pl.ds((k % 8) * 16, 16)] = (
                    ncl * n_lo + nch * n_hi)
            return rcarry

        lax.fori_loop(0, _CHUNK, row_body, 0)
        pltpu.sync_copy(
            part_v,
            out_hbm.at[pl.ds((wid * _NCHUNK + ci) * _OUT_ROWS_PER_CHUNK,
                             _OUT_ROWS_PER_CHUNK)])
        return carry

    lax.fori_loop(0, _NCHUNK, chunk_body, 0)


_sc_partial = functools.partial(
    pl.kernel,
    mesh=plsc.VectorSubcoreMesh(core_axis_name="c", subcore_axis_name="s"),
    out_type=jax.ShapeDtypeStruct((_OUT_ROWS, 128), jnp.float32),
    scratch_types=[
        pltpu.VMEM((_ROWS_W,), jnp.int32),
        pltpu.VMEM((_ROWS_W,), jnp.int32),
        pltpu.VMEM((_ROWS_W * _NNEG,), jnp.int32),
        pltpu.VMEM((_CHUNK, _D), jnp.float32),
        pltpu.VMEM((_CHUNK, _D), jnp.float32),
        pltpu.VMEM((_NEG_PER_CHUNK, _D), jnp.float32),
        pltpu.VMEM((_OUT_ROWS_PER_CHUNK, 128), jnp.float32),
        pltpu.SemaphoreType.DMA,
    ],
    compiler_params=pltpu.CompilerParams(use_tc_tiling_on_sc=False),
)(_sc_partial_body)

_BLK = 7168
_NBLK = _OUT_ROWS // _BLK


def _loss_body(p_ref, o_ref):
    i = pl.program_id(0)
    x = p_ref[...]
    lane = lax.broadcasted_iota(jnp.int32, (128, 8), 0)
    grp = lax.broadcasted_iota(jnp.int32, (128, 8), 1)
    m = jnp.where(lane // 16 == grp, 1.0, 0.0).astype(jnp.float32)
    s = jnp.dot(x, m, preferred_element_type=jnp.float32)
    ls = jnp.minimum(s, 0.0) - jnp.log1p(jnp.exp(-jnp.abs(s)))

    @pl.when(i == 0)
    def _init():
        o_ref[...] = jnp.zeros((1, 1), jnp.float32)

    o_ref[...] += jnp.sum(ls).reshape(1, 1)

    @pl.when(i == _NBLK - 1)
    def _fini():
        o_ref[...] = -o_ref[...] / _B


_loss_call = pl.pallas_call(
    _loss_body,
    grid=(_NBLK,),
    in_specs=[pl.BlockSpec((_BLK, 128), lambda i: (i, 0))],
    out_specs=pl.BlockSpec((1, 1), lambda i: (0, 0)),
    out_shape=jax.ShapeDtypeStruct((1, 1), jnp.float32),
)


def kernel(center, context, negatives, center_W, context_W):
    cidx = center.reshape(_B).astype(jnp.int32)
    pidx = context.reshape(_B).astype(jnp.int32)
    nidx = negatives.reshape(_B * _NNEG).astype(jnp.int32)
    cw4, xw4 = _conv_call(center_W.T, context_W.T)
    part = _sc_partial(cw4.reshape(_VPAD, 32), xw4.reshape(_VPAD, 32),
                       cidx, pidx, nidx)
    return _loss_call(part).reshape(())


# speedup vs baseline: 2.7221x; 1.0574x over previous
import functools

import jax
import jax.numpy as jnp
from jax import lax
from jax.experimental import pallas as pl
from jax.experimental.pallas import tpu as pltpu
from jax.experimental.pallas import tpu_sc as plsc

_B = 16384
_D = 32
_NNEG = 20
_NSC = 21
_NC, _NS = 2, 16
_NW = _NC * _NS
_ROWS_W = _B // _NW
_CHUNK = 16
_NCHUNK = _ROWS_W // _CHUNK
_NEG_PER_CHUNK = _CHUNK * _NNEG
_VEC_PER_CHUNK = _CHUNK * _NSC
_OUT_ROWS_PER_CHUNK = _VEC_PER_CHUNK * 16 // 128
_OUT_ROWS = _B * _NSC * 16 // 128

_CV = 8192
_CQ = _CV // 4
_CGRID = (1000000 + _CV - 1) // _CV
_VPAD = _CGRID * _CV


def _conv_body(a_ref, b_ref, oa_ref, ob_ref):
    for (in_ref, o_ref) in ((a_ref, oa_ref), (b_ref, ob_ref)):
        o_ref[...] = jnp.concatenate(
            [jnp.transpose(in_ref[:, pl.ds(_CQ * g, _CQ)]) for g in range(4)],
            axis=1)


_conv_call = pl.pallas_call(
    _conv_body,
    grid=(_CGRID,),
    in_specs=[pl.BlockSpec((32, _CV), lambda i: (0, i)),
              pl.BlockSpec((32, _CV), lambda i: (0, i))],
    out_specs=[pl.BlockSpec((_CQ, 128), lambda i: (i, 0)),
               pl.BlockSpec((_CQ, 128), lambda i: (i, 0))],
    out_shape=[jax.ShapeDtypeStruct((_VPAD // 4, 128), jnp.float32),
               jax.ShapeDtypeStruct((_VPAD // 4, 128), jnp.float32)],
)


def _permute_idx(ref, nvec):
    def body(k, carry):
        v = ref[pl.ds(k * 16, 16)]
        c = jnp.bitwise_and(v, _CV - 1)
        r = jnp.bitwise_and(c, _CQ - 1)
        g = jnp.right_shift(c, _CQ.bit_length() - 1)
        ref[pl.ds(k * 16, 16)] = (v - c) + jnp.left_shift(r, 2) + g
        return carry

    lax.fori_loop(0, nvec, body, 0)


def _sc_partial_body(cW_hbm, xW_hbm, cidx_hbm, pidx_hbm, nidx_hbm, out_hbm,
                     cidx_v, pidx_v, nidx_v, crow_v, prow_v, nrow_v,
                     crow2_v, prow2_v, nrow2_v, part_v, sem, sem2):
    wid = lax.axis_index("s") * _NC + lax.axis_index("c")
    base = wid * _ROWS_W
    pltpu.sync_copy(cidx_hbm.at[pl.ds(base, _ROWS_W)], cidx_v)
    pltpu.sync_copy(pidx_hbm.at[pl.ds(base, _ROWS_W)], pidx_v)
    pltpu.sync_copy(nidx_hbm.at[pl.ds(base * _NNEG, _ROWS_W * _NNEG)], nidx_v)
    _permute_idx(cidx_v, _ROWS_W // 16)
    _permute_idx(pidx_v, _ROWS_W // 16)
    _permute_idx(nidx_v, _ROWS_W * _NNEG // 16)

    bufs = ((crow_v, prow_v, nrow_v, sem), (crow2_v, prow2_v, nrow2_v, sem2))

    def fire(ci, b):
        crow, prow, nrow, bsem = bufs[b]
        pltpu.async_copy(
            cW_hbm.at[cidx_v.at[pl.ds(ci * _CHUNK, _CHUNK)]], crow, bsem)
        pltpu.async_copy(
            xW_hbm.at[pidx_v.at[pl.ds(ci * _CHUNK, _CHUNK)]], prow, bsem)
        for off, ln in ((0, 128), (128, 128), (256, 64)):
            pltpu.async_copy(
                xW_hbm.at[nidx_v.at[pl.ds(ci * _NEG_PER_CHUNK + off, ln)]],
                nrow.at[pl.ds(off, ln)], bsem)

    def drain(b):
        crow, prow, nrow, bsem = bufs[b]
        for dst in (crow, prow, nrow):
            pltpu.make_async_copy(cW_hbm.at[pl.ds(0, dst.shape[0])],
                                  dst, bsem).wait()

    def compute(ci, b):
        crow, prow, nrow, _ = bufs[b]

        def row_body(r, rcarry):
            c_lo = crow[r, pl.ds(0, 16)]
            c_hi = crow[r, pl.ds(16, 16)]
            ncl, nch = -c_lo, -c_hi
            x_lo = prow[r, pl.ds(0, 16)]
            x_hi = prow[r, pl.ds(16, 16)]
            part_v[r // 8, pl.ds((r % 8) * 16, 16)] = c_lo * x_lo + c_hi * x_hi
            for j in range(_NNEG):
                k = r * _NNEG + j
                n_lo = nrow[k, pl.ds(0, 16)]
                n_hi = nrow[k, pl.ds(16, 16)]
                part_v[2 + k // 8, pl.ds((k % 8) * 16, 16)] = (
                    ncl * n_lo + nch * n_hi)
            return rcarry

        lax.fori_loop(0, _CHUNK, row_body, 0)
        pltpu.sync_copy(
            part_v,
            out_hbm.at[pl.ds((wid * _NCHUNK + ci) * _OUT_ROWS_PER_CHUNK,
                             _OUT_ROWS_PER_CHUNK)])

    fire(0, 0)

    def outer_body(h, carry):
        ci0 = 2 * h
        fire(ci0 + 1, 1)
        drain(0)
        compute(ci0, 0)

        @pl.when(ci0 + 2 < _NCHUNK)
        def _():
            fire(ci0 + 2, 0)

        drain(1)
        compute(ci0 + 1, 1)
        return carry

    lax.fori_loop(0, _NCHUNK // 2, outer_body, 0)


_sc_partial = functools.partial(
    pl.kernel,
    mesh=plsc.VectorSubcoreMesh(core_axis_name="c", subcore_axis_name="s"),
    out_type=jax.ShapeDtypeStruct((_OUT_ROWS, 128), jnp.float32),
    scratch_types=[
        pltpu.VMEM((_ROWS_W,), jnp.int32),
        pltpu.VMEM((_ROWS_W,), jnp.int32),
        pltpu.VMEM((_ROWS_W * _NNEG,), jnp.int32),
        pltpu.VMEM((_CHUNK, _D), jnp.float32),
        pltpu.VMEM((_CHUNK, _D), jnp.float32),
        pltpu.VMEM((_NEG_PER_CHUNK, _D), jnp.float32),
        pltpu.VMEM((_CHUNK, _D), jnp.float32),
        pltpu.VMEM((_CHUNK, _D), jnp.float32),
        pltpu.VMEM((_NEG_PER_CHUNK, _D), jnp.float32),
        pltpu.VMEM((_OUT_ROWS_PER_CHUNK, 128), jnp.float32),
        pltpu.SemaphoreType.DMA,
        pltpu.SemaphoreType.DMA,
    ],
    compiler_params=pltpu.CompilerParams(use_tc_tiling_on_sc=False),
)(_sc_partial_body)

_BLK = 7168
_NBLK = _OUT_ROWS // _BLK


def _loss_body(p_ref, o_ref):
    i = pl.program_id(0)
    x = p_ref[...]
    lane = lax.broadcasted_iota(jnp.int32, (128, 8), 0)
    grp = lax.broadcasted_iota(jnp.int32, (128, 8), 1)
    m = jnp.where(lane // 16 == grp, 1.0, 0.0).astype(jnp.float32)
    s = jnp.dot(x, m, preferred_element_type=jnp.float32)
    ls = jnp.minimum(s, 0.0) - jnp.log1p(jnp.exp(-jnp.abs(s)))

    @pl.when(i == 0)
    def _init():
        o_ref[...] = jnp.zeros((1, 1), jnp.float32)

    o_ref[...] += jnp.sum(ls).reshape(1, 1)

    @pl.when(i == _NBLK - 1)
    def _fini():
        o_ref[...] = -o_ref[...] / _B


_loss_call = pl.pallas_call(
    _loss_body,
    grid=(_NBLK,),
    in_specs=[pl.BlockSpec((_BLK, 128), lambda i: (i, 0))],
    out_specs=pl.BlockSpec((1, 1), lambda i: (0, 0)),
    out_shape=jax.ShapeDtypeStruct((1, 1), jnp.float32),
)


def kernel(center, context, negatives, center_W, context_W):
    cidx = center.reshape(_B).astype(jnp.int32)
    pidx = context.reshape(_B).astype(jnp.int32)
    nidx = negatives.reshape(_B * _NNEG).astype(jnp.int32)
    cw4, xw4 = _conv_call(center_W.T, context_W.T)
    part = _sc_partial(cw4.reshape(_VPAD, 32), xw4.reshape(_VPAD, 32),
                       cidx, pidx, nidx)
    return _loss_call(part).reshape(())


# speedup vs baseline: 2.7247x; 1.0009x over previous
import functools

import jax
import jax.numpy as jnp
from jax import lax
from jax.experimental import pallas as pl
from jax.experimental.pallas import tpu as pltpu
from jax.experimental.pallas import tpu_sc as plsc

_B = 16384
_D = 32
_NNEG = 20
_NSC = 21
_NC, _NS = 2, 16
_NW = _NC * _NS
_ROWS_W = _B // _NW
_CHUNK = 32
_NCHUNK = _ROWS_W // _CHUNK
_NEG_PER_CHUNK = _CHUNK * _NNEG
_VEC_PER_CHUNK = _CHUNK * _NSC
_OUT_ROWS_PER_CHUNK = _VEC_PER_CHUNK * 16 // 128
_OUT_ROWS = _B * _NSC * 16 // 128

_CV = 8192
_CQ = _CV // 4
_CGRID = (1000000 + _CV - 1) // _CV
_VPAD = _CGRID * _CV


def _conv_body(a_ref, b_ref, oa_ref, ob_ref):
    for (in_ref, o_ref) in ((a_ref, oa_ref), (b_ref, ob_ref)):
        o_ref[...] = jnp.concatenate(
            [jnp.transpose(in_ref[:, pl.ds(_CQ * g, _CQ)]) for g in range(4)],
            axis=1)


_conv_call = pl.pallas_call(
    _conv_body,
    grid=(_CGRID,),
    in_specs=[pl.BlockSpec((32, _CV), lambda i: (0, i)),
              pl.BlockSpec((32, _CV), lambda i: (0, i))],
    out_specs=[pl.BlockSpec((_CQ, 128), lambda i: (i, 0)),
               pl.BlockSpec((_CQ, 128), lambda i: (i, 0))],
    out_shape=[jax.ShapeDtypeStruct((_VPAD // 4, 128), jnp.float32),
               jax.ShapeDtypeStruct((_VPAD // 4, 128), jnp.float32)],
)


def _permute_idx(ref, nvec):
    def body(k, carry):
        v = ref[pl.ds(k * 16, 16)]
        c = jnp.bitwise_and(v, _CV - 1)
        r = jnp.bitwise_and(c, _CQ - 1)
        g = jnp.right_shift(c, _CQ.bit_length() - 1)
        ref[pl.ds(k * 16, 16)] = (v - c) + jnp.left_shift(r, 2) + g
        return carry

    lax.fori_loop(0, nvec, body, 0)


def _sc_partial_body(cW_hbm, xW_hbm, cidx_hbm, pidx_hbm, nidx_hbm, out_hbm,
                     cidx_v, pidx_v, nidx_v, crow_v, prow_v, nrow_v,
                     crow2_v, prow2_v, nrow2_v, part_v, sem, sem2):
    wid = lax.axis_index("s") * _NC + lax.axis_index("c")
    base = wid * _ROWS_W
    pltpu.sync_copy(cidx_hbm.at[pl.ds(base, _ROWS_W)], cidx_v)
    pltpu.sync_copy(pidx_hbm.at[pl.ds(base, _ROWS_W)], pidx_v)
    pltpu.sync_copy(nidx_hbm.at[pl.ds(base * _NNEG, _ROWS_W * _NNEG)], nidx_v)
    _permute_idx(cidx_v, _ROWS_W // 16)
    _permute_idx(pidx_v, _ROWS_W // 16)
    _permute_idx(nidx_v, _ROWS_W * _NNEG // 16)

    bufs = ((crow_v, prow_v, nrow_v, sem), (crow2_v, prow2_v, nrow2_v, sem2))

    def fire(ci, b):
        crow, prow, nrow, bsem = bufs[b]
        pltpu.async_copy(
            cW_hbm.at[cidx_v.at[pl.ds(ci * _CHUNK, _CHUNK)]], crow, bsem)
        pltpu.async_copy(
            xW_hbm.at[pidx_v.at[pl.ds(ci * _CHUNK, _CHUNK)]], prow, bsem)
        for off in range(0, _NEG_PER_CHUNK, 128):
            ln = min(128, _NEG_PER_CHUNK - off)
            pltpu.async_copy(
                xW_hbm.at[nidx_v.at[pl.ds(ci * _NEG_PER_CHUNK + off, ln)]],
                nrow.at[pl.ds(off, ln)], bsem)

    def drain(b):
        crow, prow, nrow, bsem = bufs[b]
        for dst in (crow, prow, nrow):
            pltpu.make_async_copy(cW_hbm.at[pl.ds(0, dst.shape[0])],
                                  dst, bsem).wait()

    def compute(ci, b):
        crow, prow, nrow, _ = bufs[b]

        def row_body(r, rcarry):
            c_lo = crow[r, pl.ds(0, 16)]
            c_hi = crow[r, pl.ds(16, 16)]
            ncl, nch = -c_lo, -c_hi
            x_lo = prow[r, pl.ds(0, 16)]
            x_hi = prow[r, pl.ds(16, 16)]
            part_v[r // 8, pl.ds((r % 8) * 16, 16)] = c_lo * x_lo + c_hi * x_hi
            for j in range(_NNEG):
                k = r * _NNEG + j
                n_lo = nrow[k, pl.ds(0, 16)]
                n_hi = nrow[k, pl.ds(16, 16)]
                part_v[2 + k // 8, pl.ds((k % 8) * 16, 16)] = (
                    ncl * n_lo + nch * n_hi)
            return rcarry

        lax.fori_loop(0, _CHUNK, row_body, 0)
        pltpu.sync_copy(
            part_v,
            out_hbm.at[pl.ds((wid * _NCHUNK + ci) * _OUT_ROWS_PER_CHUNK,
                             _OUT_ROWS_PER_CHUNK)])

    fire(0, 0)

    def outer_body(h, carry):
        ci0 = 2 * h
        fire(ci0 + 1, 1)
        drain(0)
        compute(ci0, 0)

        @pl.when(ci0 + 2 < _NCHUNK)
        def _():
            fire(ci0 + 2, 0)

        drain(1)
        compute(ci0 + 1, 1)
        return carry

    lax.fori_loop(0, _NCHUNK // 2, outer_body, 0)


_sc_partial = functools.partial(
    pl.kernel,
    mesh=plsc.VectorSubcoreMesh(core_axis_name="c", subcore_axis_name="s"),
    out_type=jax.ShapeDtypeStruct((_OUT_ROWS, 128), jnp.float32),
    scratch_types=[
        pltpu.VMEM((_ROWS_W,), jnp.int32),
        pltpu.VMEM((_ROWS_W,), jnp.int32),
        pltpu.VMEM((_ROWS_W * _NNEG,), jnp.int32),
        pltpu.VMEM((_CHUNK, _D), jnp.float32),
        pltpu.VMEM((_CHUNK, _D), jnp.float32),
        pltpu.VMEM((_NEG_PER_CHUNK, _D), jnp.float32),
        pltpu.VMEM((_CHUNK, _D), jnp.float32),
        pltpu.VMEM((_CHUNK, _D), jnp.float32),
        pltpu.VMEM((_NEG_PER_CHUNK, _D), jnp.float32),
        pltpu.VMEM((_OUT_ROWS_PER_CHUNK, 128), jnp.float32),
        pltpu.SemaphoreType.DMA,
        pltpu.SemaphoreType.DMA,
    ],
    compiler_params=pltpu.CompilerParams(use_tc_tiling_on_sc=False),
)(_sc_partial_body)

_BLK = 7168
_NBLK = _OUT_ROWS // _BLK


def _loss_body(p_ref, o_ref):
    i = pl.program_id(0)
    x = p_ref[...]
    lane = lax.broadcasted_iota(jnp.int32, (128, 8), 0)
    grp = lax.broadcasted_iota(jnp.int32, (128, 8), 1)
    m = jnp.where(lane // 16 == grp, 1.0, 0.0).astype(jnp.float32)
    s = jnp.dot(x, m, preferred_element_type=jnp.float32)
    ls = jnp.minimum(s, 0.0) - jnp.log1p(jnp.exp(-jnp.abs(s)))

    @pl.when(i == 0)
    def _init():
        o_ref[...] = jnp.zeros((1, 1), jnp.float32)

    o_ref[...] += jnp.sum(ls).reshape(1, 1)

    @pl.when(i == _NBLK - 1)
    def _fini():
        o_ref[...] = -o_ref[...] / _B


_loss_call = pl.pallas_call(
    _loss_body,
    grid=(_NBLK,),
    in_specs=[pl.BlockSpec((_BLK, 128), lambda i: (i, 0))],
    out_specs=pl.BlockSpec((1, 1), lambda i: (0, 0)),
    out_shape=jax.ShapeDtypeStruct((1, 1), jnp.float32),
)


def kernel(center, context, negatives, center_W, context_W):
    cidx = center.reshape(_B).astype(jnp.int32)
    pidx = context.reshape(_B).astype(jnp.int32)
    nidx = negatives.reshape(_B * _NNEG).astype(jnp.int32)
    cw4, xw4 = _conv_call(center_W.T, context_W.T)
    part = _sc_partial(cw4.reshape(_VPAD, 32), xw4.reshape(_VPAD, 32),
                       cidx, pidx, nidx)
    return _loss_call(part).reshape(())
